# baseline (speedup 1.0000x reference)
import numpy as np
import jax
import jax.numpy as jnp
from jax import lax
from jax.experimental import pallas as pl
from jax.experimental.pallas import tpu as pltpu, tpu_sc as plsc

F = 9
V = 124
D = 128
N = 50000
NC, NS = 2, 16
NW = NC * NS
NB = 32
BLK = 49
NPW = NB * BLK
TAIL = N - (NW - 1) * NPW

_PERM = np.empty((D,), dtype=np.int32)
for _g in range(D // 32):
    for _j in range(16):
        _PERM[_g * 32 + 2 * _j] = _g * 32 + _j
        _PERM[_g * 32 + 2 * _j + 1] = _g * 32 + 16 + _j


def _body(idx_hbm, table_hbm, out_hbm, idx_v, buf, out_v, sg0, sg1, ss0, ss1):
    wid = lax.axis_index("s") * NC + lax.axis_index("c")
    base0 = wid * NPW
    sg = (sg0, sg1)
    ss = (ss0, ss1)

    @pl.when(wid < NW - 1)
    def _():
        for f in range(F):
            pltpu.sync_copy(idx_hbm.at[f, pl.ds(base0, NPW)], idx_v.at[f])

    @pl.when(wid == NW - 1)
    def _():
        for f in range(F):
            pltpu.sync_copy(idx_hbm.at[f, pl.ds(base0, TAIL)],
                            idx_v.at[f, pl.ds(0, TAIL)])
            idx_v[f, pl.ds(TAIL, 16)] = jnp.zeros((16,), jnp.int32)

    def add_off(c, carry):
        sl = pl.ds(c * 16, 16)
        for f in range(1, F):
            idx_v[f, sl] = idx_v[f, sl] + f * V
        return carry
    lax.fori_loop(0, TAIL // 16 + 1, add_off, 0)

    @pl.when(wid < NW - 1)
    def _():
        def add_off_tail(c, carry):
            sl = pl.ds(c * 16, 16)
            for f in range(1, F):
                idx_v[f, sl] = idx_v[f, sl] + f * V
            return carry
        lax.fori_loop(TAIL // 16 + 1, NPW // 16, add_off_tail, 0)

    def gathers(j, b):
        return [
            pltpu.make_async_copy(table_hbm.at[idx_v.at[f, pl.ds(j * NB, NB)]],
                                  buf.at[b, f], sg[b])
            for f in range(F)
        ]

    def fire_gathers(j, b):
        for f in range(F):
            pltpu.async_copy(table_hbm.at[idx_v.at[f, pl.ds(j * NB, NB)]],
                             buf.at[b, f], sg[b])

    def full_desc(j, b):
        return pltpu.make_async_copy(out_v.at[b],
                                     out_hbm.at[pl.ds(base0 + j * NB, NB)],
                                     ss[b])

    def part_desc(j, b):
        return pltpu.make_async_copy(out_v.at[b, pl.ds(0, 16)],
                                     out_hbm.at[pl.ds(base0 + j * NB, 16)],
                                     ss[b])

    fire_gathers(0, 0)
    fire_gathers(1, 1)

    def conds(j):
        base = base0 + j * NB
        valid = jnp.logical_and(j < BLK, base < N)
        full = jnp.logical_and(j < BLK, base + NB <= N)
        part = jnp.logical_and(valid, jnp.logical_not(full))
        return valid, full, part

    def pair(jp, c):
        for b in range(2):
            j = jp * 2 + b
            valid, full, part = conds(j)
            pv, pf, pp = conds(j - 2)

            @pl.when(jnp.logical_and(j >= 2, pf))
            def _():
                full_desc(j - 2, b).wait()

            @pl.when(jnp.logical_and(j >= 2, pp))
            def _():
                part_desc(j - 2, b).wait()

            @pl.when(valid)
            def _():
                for dsc in gathers(j, b):
                    dsc.wait()

                def acc_row(r, cc):
                    for ch in range(D // 32):
                        sl = pl.ds(ch * 32, 32)
                        a, bb = plsc.unpack(
                            buf[b, 0, r, sl],
                            format=plsc.PackFormat.INTERLEAVED)
                        for f in range(1, F):
                            af, bf = plsc.unpack(
                                buf[b, f, r, sl],
                                format=plsc.PackFormat.INTERLEAVED)
                            a = a + af
                            bb = bb + bf
                        out_v[b, r, pl.ds(ch * 32, 16)] = a
                        out_v[b, r, pl.ds(ch * 32 + 16, 16)] = bb
                    return cc
                lax.fori_loop(0, NB, acc_row, 0)

            nv, _, _ = conds(j + 2)

            @pl.when(nv)
            def _():
                fire_gathers(j + 2, b)

            @pl.when(full)
            def _():
                pltpu.async_copy(out_v.at[b],
                                 out_hbm.at[pl.ds(base0 + j * NB, NB)], ss[b])

            @pl.when(part)
            def _():
                pltpu.async_copy(out_v.at[b, pl.ds(0, 16)],
                                 out_hbm.at[pl.ds(base0 + j * NB, 16)], ss[b])
        return c

    lax.fori_loop(0, (BLK + 1) // 2 + 1, pair, 0)


@jax.jit
def _sc_embed(node_features, flat_tables):
    return pl.kernel(
        _body,
        out_type=jax.ShapeDtypeStruct((N, D), jnp.float32),
        mesh=plsc.VectorSubcoreMesh(core_axis_name="c", subcore_axis_name="s"),
        scratch_types=[
            pltpu.VMEM((F, NPW), jnp.int32),
            pltpu.VMEM((2, F, NB, D), jnp.bfloat16),
            pltpu.VMEM((2, NB, D), jnp.float32),
            pltpu.SemaphoreType.DMA,
            pltpu.SemaphoreType.DMA,
            pltpu.SemaphoreType.DMA,
            pltpu.SemaphoreType.DMA,
        ],
        compiler_params=pltpu.CompilerParams(use_tc_tiling_on_sc=False,
                                             needs_layout_passes=False),
    )(node_features, flat_tables)


def kernel(node_features, tables):
    flat_tables = tables.reshape(F * V, D)[:, _PERM].astype(jnp.bfloat16)
    return _sc_embed(node_features, flat_tables)


# speedup vs baseline: 1.0055x; 1.0055x over previous
import numpy as np
import jax
import jax.numpy as jnp
from jax import lax
from jax.experimental import pallas as pl
from jax.experimental.pallas import tpu as pltpu, tpu_sc as plsc

F = 9
V = 124
D = 128
N = 50000
NC, NS = 2, 16
NW = NC * NS
NB = 16
BLK = 98
NPW = NB * BLK
TAIL = N - (NW - 1) * NPW
NBUF = 4

_PERM = np.empty((D,), dtype=np.int32)
for _g in range(D // 32):
    for _j in range(16):
        _PERM[_g * 32 + 2 * _j] = _g * 32 + _j
        _PERM[_g * 32 + 2 * _j + 1] = _g * 32 + 16 + _j


def _body(idx_hbm, table_hbm, out_hbm, idx_v, buf, out_v, *sems):
    wid = lax.axis_index("s") * NC + lax.axis_index("c")
    base0 = wid * NPW
    sg = sems[:NBUF]
    ss = sems[NBUF:]

    @pl.when(wid < NW - 1)
    def _():
        for f in range(F):
            pltpu.sync_copy(idx_hbm.at[f, pl.ds(base0, NPW)], idx_v.at[f])

    @pl.when(wid == NW - 1)
    def _():
        for f in range(F):
            pltpu.sync_copy(idx_hbm.at[f, pl.ds(base0, TAIL)],
                            idx_v.at[f, pl.ds(0, TAIL)])

    def add_off(c, carry):
        sl = pl.ds(c * 16, 16)
        for f in range(1, F):
            idx_v[f, sl] = idx_v[f, sl] + f * V
        return carry
    lax.fori_loop(0, TAIL // 16, add_off, 0)

    @pl.when(wid < NW - 1)
    def _():
        def add_off_tail(c, carry):
            sl = pl.ds(c * 16, 16)
            for f in range(1, F):
                idx_v[f, sl] = idx_v[f, sl] + f * V
            return carry
        lax.fori_loop(TAIL // 16, NPW // 16, add_off_tail, 0)

    def valid(j):
        return jnp.logical_and(j < BLK, base0 + j * NB < N)

    def gathers(j, b):
        return [
            pltpu.make_async_copy(table_hbm.at[idx_v.at[f, pl.ds(j * NB, NB)]],
                                  buf.at[b, f], sg[b])
            for f in range(F)
        ]

    def fire_gathers(j, b):
        for f in range(F):
            pltpu.async_copy(table_hbm.at[idx_v.at[f, pl.ds(j * NB, NB)]],
                             buf.at[b, f], sg[b])

    def store_desc(j, b):
        return pltpu.make_async_copy(out_v.at[b],
                                     out_hbm.at[pl.ds(base0 + j * NB, NB)],
                                     ss[b])

    for j0 in range(NBUF - 1):
        fire_gathers(j0, j0)

    def group(jg, c):
        for b in range(NBUF):
            j = jg * NBUF + b

            @pl.when(jnp.logical_and(j >= NBUF, valid(j - NBUF)))
            def _():
                store_desc(j - NBUF, b).wait()

            @pl.when(valid(j + NBUF - 1))
            def _():
                fire_gathers(j + NBUF - 1, (b + NBUF - 1) % NBUF)

            @pl.when(valid(j))
            def _():
                for dsc in gathers(j, b):
                    dsc.wait()

                def acc_row(r, cc):
                    for ch in range(D // 32):
                        sl = pl.ds(ch * 32, 32)
                        a, bb = plsc.unpack(
                            buf[b, 0, r, sl],
                            format=plsc.PackFormat.INTERLEAVED)
                        for f in range(1, F):
                            af, bf = plsc.unpack(
                                buf[b, f, r, sl],
                                format=plsc.PackFormat.INTERLEAVED)
                            a = a + af
                            bb = bb + bf
                        out_v[b, r, pl.ds(ch * 32, 16)] = a
                        out_v[b, r, pl.ds(ch * 32 + 16, 16)] = bb
                    return cc
                lax.fori_loop(0, NB, acc_row, 0)

                pltpu.async_copy(out_v.at[b],
                                 out_hbm.at[pl.ds(base0 + j * NB, NB)], ss[b])
        return c

    lax.fori_loop(0, BLK // NBUF + 2, group, 0)


@jax.jit
def _sc_embed(node_features, flat_tables):
    return pl.kernel(
        _body,
        out_type=jax.ShapeDtypeStruct((N, D), jnp.float32),
        mesh=plsc.VectorSubcoreMesh(core_axis_name="c", subcore_axis_name="s"),
        scratch_types=[
            pltpu.VMEM((F, NPW), jnp.int32),
            pltpu.VMEM((NBUF, F, NB, D), jnp.bfloat16),
            pltpu.VMEM((NBUF, NB, D), jnp.float32),
        ] + [pltpu.SemaphoreType.DMA] * (2 * NBUF),
        compiler_params=pltpu.CompilerParams(use_tc_tiling_on_sc=False,
                                             needs_layout_passes=False),
    )(node_features, flat_tables)


def kernel(node_features, tables):
    flat_tables = tables.reshape(F * V, D)[:, _PERM].astype(jnp.bfloat16)
    return _sc_embed(node_features, flat_tables)
